# baseline (speedup 1.0000x reference)
import functools

import jax
import jax.numpy as jnp
from jax import lax
from jax.experimental import pallas as pl
from jax.experimental.pallas import tpu as pltpu
from jax.experimental.pallas import tpu_sc as plsc

B = 16384
F = 26
V = 100000
K = 16

NC = 2
NS = 16
NW = NC * NS

ROWS_PER_W = B // NW
CH = 64
NCH = ROWS_PER_W // CH
IDX_PER_CH = CH * F
NDMA = IDX_PER_CH // 128

BG = 1792
WIN = 8 * BG
NGB = 7
G = NGB * BG
ROW_PITCH = G * 8

_mesh = plsc.VectorSubcoreMesh(
    core_axis_name="c", subcore_axis_name="s", num_cores=NC, num_subcores=NS
)


def _relayout_body(t2_ref, out_ref):
    xw = t2_ref[0]
    xcat = jnp.concatenate(
        [xw[:, m * BG:(m + 1) * BG] for m in range(8)], axis=0)
    out_ref[0] = jnp.transpose(xcat, (1, 0))


def _relayout(t2):
    return pl.pallas_call(
        _relayout_body,
        grid=(F, NGB),
        in_specs=[pl.BlockSpec((1, K, WIN), lambda f, gb: (f, 0, gb))],
        out_specs=pl.BlockSpec((1, BG, 128), lambda f, gb: (f, gb, 0)),
        out_shape=jax.ShapeDtypeStruct((F, G, 128), jnp.float32),
    )(t2)


_SC_SCRATCH = [
    pltpu.VMEM((NCH * NDMA, 128), jnp.int32),
    pltpu.VMEM((IDX_PER_CH, K), jnp.float32),
    pltpu.VMEM((CH, K), jnp.float32),
    pltpu.VMEM((CH, K), jnp.float32),
    pltpu.VMEM((K,), jnp.float32),
    pltpu.SemaphoreType.DMA,
]


@functools.partial(
    pl.kernel,
    out_type=jax.ShapeDtypeStruct((B, K), jnp.float32),
    mesh=_mesh,
    compiler_params=pltpu.CompilerParams(use_tc_tiling_on_sc=False),
    scratch_types=_SC_SCRATCH,
)
def _sum_kernel(idx_hbm, sm_tab, smb_hbm, out_hbm,
                idx_v, rows_v, out_v, unused_v, bias_v, sem):
    wid = lax.axis_index("s") * NC + lax.axis_index("c")
    pltpu.sync_copy(smb_hbm, bias_v)
    pltpu.sync_copy(
        idx_hbm.at[pl.ds(wid * (NCH * NDMA), NCH * NDMA)], idx_v)

    def chunk_body(c, carry):
        copies = []
        for j in range(NDMA):
            copies.append(pltpu.async_copy(
                sm_tab.at[idx_v.at[c * NDMA + j]],
                rows_v.at[pl.ds(j * 128, 128)], sem))
        for cp in copies:
            cp.wait()

        def row_body(r, rcarry):
            base = r * F
            s = bias_v[...]
            for f in range(F):
                s = s + rows_v[base + f, :]
            out_v[r, :] = s
            return rcarry

        lax.fori_loop(0, CH, row_body, 0)
        pltpu.sync_copy(
            out_v, out_hbm.at[pl.ds(wid * ROWS_PER_W + c * CH, CH)])
        return carry

    lax.fori_loop(0, NCH, chunk_body, 0)


@functools.partial(
    pl.kernel,
    out_type=jax.ShapeDtypeStruct((B, K), jnp.float32),
    mesh=_mesh,
    compiler_params=pltpu.CompilerParams(use_tc_tiling_on_sc=False),
    scratch_types=_SC_SCRATCH,
)
def _combine_kernel(idx_hbm, sg_tab, sgb_hbm, ssum_hbm, out_hbm,
                    idx_v, rows_v, out_v, ss_v, bias_v, sem):
    wid = lax.axis_index("s") * NC + lax.axis_index("c")
    pltpu.sync_copy(sgb_hbm, bias_v)
    pltpu.sync_copy(
        idx_hbm.at[pl.ds(wid * (NCH * NDMA), NCH * NDMA)], idx_v)

    def chunk_body(c, carry):
        copies = []
        for j in range(NDMA):
            copies.append(pltpu.async_copy(
                sg_tab.at[idx_v.at[c * NDMA + j]],
                rows_v.at[pl.ds(j * 128, 128)], sem))
        pltpu.sync_copy(
            ssum_hbm.at[pl.ds(wid * ROWS_PER_W + c * CH, CH)], ss_v)
        for cp in copies:
            cp.wait()

        lanes = lax.broadcasted_iota(jnp.int32, (K,), 0)

        def lane_sum(x):
            for sh in (8, 4, 2, 1):
                x = x + x.at[lanes ^ sh].get(mode="promise_in_bounds")
            return x

        def row_body(r, rcarry):
            base = r * F
            g = bias_v[...]
            for f in range(F):
                g = g + rows_v[base + f, :]
            e = jnp.exp(ss_v[r, :])
            sig = 1.0 / (1.0 + jnp.exp(g))
            val = lane_sum(e * sig) / lane_sum(e)
            out_v[r, :] = val
            return rcarry

        lax.fori_loop(0, CH, row_body, 0)
        pltpu.sync_copy(
            out_v, out_hbm.at[pl.ds(wid * ROWS_PER_W + c * CH, CH)])
        return carry

    lax.fori_loop(0, NCH, chunk_body, 0)


def kernel(indices, softmax_W, sigmoid_W, softmax_bias, sigmoid_bias):
    col_off = (jnp.arange(F, dtype=jnp.int32) * ROW_PITCH)[None, :]
    gb = indices // WIN
    o = indices % WIN
    perm = (gb * BG + o % BG) * 8 + o // BG
    flat_idx = (perm + col_off).reshape(B * F // 128, 128)
    sm_tab = _relayout(jnp.transpose(softmax_W, (0, 2, 1))).reshape(F * ROW_PITCH, K)
    s_sums = _sum_kernel(flat_idx, sm_tab, softmax_bias)
    sg_tab = _relayout(jnp.transpose(sigmoid_W, (0, 2, 1))).reshape(F * ROW_PITCH, K)
    logits = _combine_kernel(flat_idx, sg_tab, sigmoid_bias, s_sums)
    return logits[:, :1]


# speedup vs baseline: 1.0243x; 1.0243x over previous
import functools

import jax
import jax.numpy as jnp
from jax import lax
from jax.experimental import pallas as pl
from jax.experimental.pallas import tpu as pltpu
from jax.experimental.pallas import tpu_sc as plsc

B = 16384
F = 26
V = 100000
K = 16

NC = 2
NS = 16
NW = NC * NS

ROWS_PER_W = B // NW
CH = 128
NCH = ROWS_PER_W // CH
IDX_PER_CH = CH * F
NDMA = IDX_PER_CH // 128

G = 12544
BG = 1792
NGB = G // BG
ROW_PITCH = G * 8

_mesh = plsc.VectorSubcoreMesh(
    core_axis_name="c", subcore_axis_name="s", num_cores=NC, num_subcores=NS
)


def _relayout_body(t2_ref, out_ref):
    gb = pl.program_id(1)
    pieces = []
    for m in range(8):
        start = pl.multiple_of(m * G + gb * BG, 128)
        pieces.append(t2_ref[0, :, pl.ds(start, BG)])
    xcat = jnp.concatenate(pieces, axis=0)
    out_ref[0] = jnp.transpose(xcat, (1, 0))


def _relayout(t2):
    return pl.pallas_call(
        _relayout_body,
        grid=(F, NGB),
        in_specs=[pl.BlockSpec((1, K, V), lambda f, gb: (f, 0, 0))],
        out_specs=pl.BlockSpec((1, BG, 128), lambda f, gb: (f, gb, 0)),
        out_shape=jax.ShapeDtypeStruct((F, G, 128), jnp.float32),
    )(t2)


_SC_SCRATCH = [
    pltpu.VMEM((NCH * NDMA, 128), jnp.int32),
    pltpu.VMEM((IDX_PER_CH, K), jnp.float32),
    pltpu.VMEM((CH, K), jnp.float32),
    pltpu.VMEM((CH, K), jnp.float32),
    pltpu.VMEM((K,), jnp.float32),
    pltpu.SemaphoreType.DMA,
]


@functools.partial(
    pl.kernel,
    out_type=jax.ShapeDtypeStruct((B, K), jnp.float32),
    mesh=_mesh,
    compiler_params=pltpu.CompilerParams(use_tc_tiling_on_sc=False),
    scratch_types=_SC_SCRATCH,
)
def _sum_kernel(idx_hbm, sm_tab, smb_hbm, out_hbm,
                idx_v, rows_v, out_v, unused_v, bias_v, sem):
    wid = lax.axis_index("s") * NC + lax.axis_index("c")
    pltpu.sync_copy(smb_hbm, bias_v)
    pltpu.sync_copy(
        idx_hbm.at[pl.ds(wid * (NCH * NDMA), NCH * NDMA)], idx_v)

    def chunk_body(c, carry):
        copies = []
        for j in range(NDMA):
            copies.append(pltpu.async_copy(
                sm_tab.at[idx_v.at[c * NDMA + j]],
                rows_v.at[pl.ds(j * 128, 128)], sem))
        for cp in copies:
            cp.wait()

        def row_body(r, rcarry):
            base = r * F
            s = bias_v[...]
            for f in range(F):
                s = s + rows_v[base + f, :]
            out_v[r, :] = s
            return rcarry

        lax.fori_loop(0, CH, row_body, 0)
        pltpu.sync_copy(
            out_v, out_hbm.at[pl.ds(wid * ROWS_PER_W + c * CH, CH)])
        return carry

    lax.fori_loop(0, NCH, chunk_body, 0)


@functools.partial(
    pl.kernel,
    out_type=jax.ShapeDtypeStruct((B, K), jnp.float32),
    mesh=_mesh,
    compiler_params=pltpu.CompilerParams(use_tc_tiling_on_sc=False),
    scratch_types=_SC_SCRATCH,
)
def _combine_kernel(idx_hbm, sg_tab, sgb_hbm, ssum_hbm, out_hbm,
                    idx_v, rows_v, out_v, ss_v, bias_v, sem):
    wid = lax.axis_index("s") * NC + lax.axis_index("c")
    pltpu.sync_copy(sgb_hbm, bias_v)
    pltpu.sync_copy(
        idx_hbm.at[pl.ds(wid * (NCH * NDMA), NCH * NDMA)], idx_v)

    def chunk_body(c, carry):
        copies = []
        for j in range(NDMA):
            copies.append(pltpu.async_copy(
                sg_tab.at[idx_v.at[c * NDMA + j]],
                rows_v.at[pl.ds(j * 128, 128)], sem))
        pltpu.sync_copy(
            ssum_hbm.at[pl.ds(wid * ROWS_PER_W + c * CH, CH)], ss_v)
        for cp in copies:
            cp.wait()

        lanes = lax.broadcasted_iota(jnp.int32, (K,), 0)

        def lane_sum(x):
            for sh in (8, 4, 2, 1):
                x = x + x.at[lanes ^ sh].get(mode="promise_in_bounds")
            return x

        def row_body(r, rcarry):
            base = r * F
            g = bias_v[...]
            for f in range(F):
                g = g + rows_v[base + f, :]
            e = jnp.exp(ss_v[r, :])
            sig = 1.0 / (1.0 + jnp.exp(g))
            val = lane_sum(e * sig) / lane_sum(e)
            out_v[r, :] = val
            return rcarry

        lax.fori_loop(0, CH, row_body, 0)
        pltpu.sync_copy(
            out_v, out_hbm.at[pl.ds(wid * ROWS_PER_W + c * CH, CH)])
        return carry

    lax.fori_loop(0, NCH, chunk_body, 0)


def kernel(indices, softmax_W, sigmoid_W, softmax_bias, sigmoid_bias):
    col_off = (jnp.arange(F, dtype=jnp.int32) * ROW_PITCH)[None, :]
    perm = (indices % G) * 8 + indices // G
    flat_idx = (perm + col_off).reshape(B * F // 128, 128)
    sm_tab = _relayout(jnp.transpose(softmax_W, (0, 2, 1))).reshape(F * ROW_PITCH, K)
    s_sums = _sum_kernel(flat_idx, sm_tab, softmax_bias)
    sg_tab = _relayout(jnp.transpose(sigmoid_W, (0, 2, 1))).reshape(F * ROW_PITCH, K)
    logits = _combine_kernel(flat_idx, sg_tab, sigmoid_bias, s_sums)
    return logits[:, :1]


# speedup vs baseline: 1.4921x; 1.4567x over previous
import functools

import jax
import jax.numpy as jnp
from jax import lax
from jax.experimental import pallas as pl
from jax.experimental.pallas import tpu as pltpu
from jax.experimental.pallas import tpu_sc as plsc

B = 16384
F = 26
V = 100000
K = 16

NC = 2
NS = 16
NW = NC * NS

ROWS_PER_W = B // NW
CH = 128
NCH = ROWS_PER_W // CH
IDX_PER_CH = CH * F
NDMA = IDX_PER_CH // 128

G = 12544
BG = G
NGB = G // BG
ROW_PITCH = G * 8

_mesh = plsc.VectorSubcoreMesh(
    core_axis_name="c", subcore_axis_name="s", num_cores=NC, num_subcores=NS
)


def _relayout_body(t2_ref, out_ref):
    gb = pl.program_id(1)
    pieces = []
    for m in range(8):
        start = pl.multiple_of(m * G + gb * BG, 128)
        pieces.append(t2_ref[0, :, pl.ds(start, BG)])
    xcat = jnp.concatenate(pieces, axis=0)
    out_ref[0] = jnp.transpose(xcat, (1, 0))


def _relayout(t2):
    return pl.pallas_call(
        _relayout_body,
        grid=(F, NGB),
        in_specs=[pl.BlockSpec((1, K, V), lambda f, gb: (f, 0, 0))],
        out_specs=pl.BlockSpec((1, BG, 128), lambda f, gb: (f, gb, 0)),
        out_shape=jax.ShapeDtypeStruct((F, G, 128), jnp.float32),
    )(t2)


_SC_SCRATCH = [
    pltpu.VMEM((NCH * NDMA, 128), jnp.int32),
    pltpu.VMEM((IDX_PER_CH, K), jnp.float32),
    pltpu.VMEM((CH, K), jnp.float32),
    pltpu.VMEM((CH, K), jnp.float32),
    pltpu.VMEM((K,), jnp.float32),
    pltpu.SemaphoreType.DMA,
]


@functools.partial(
    pl.kernel,
    out_type=jax.ShapeDtypeStruct((B, K), jnp.float32),
    mesh=_mesh,
    compiler_params=pltpu.CompilerParams(use_tc_tiling_on_sc=False),
    scratch_types=_SC_SCRATCH,
)
def _sum_kernel(idx_hbm, sm_tab, smb_hbm, out_hbm,
                idx_v, rows_v, out_v, unused_v, bias_v, sem):
    wid = lax.axis_index("s") * NC + lax.axis_index("c")
    pltpu.sync_copy(smb_hbm, bias_v)
    pltpu.sync_copy(
        idx_hbm.at[pl.ds(wid * (NCH * NDMA), NCH * NDMA)], idx_v)

    def chunk_body(c, carry):
        copies = []
        for j in range(NDMA):
            copies.append(pltpu.async_copy(
                sm_tab.at[idx_v.at[c * NDMA + j]],
                rows_v.at[pl.ds(j * 128, 128)], sem))
        for cp in copies:
            cp.wait()

        def row_body(r, rcarry):
            base = r * F
            s = bias_v[...]
            for f in range(F):
                s = s + rows_v[base + f, :]
            out_v[r, :] = s
            return rcarry

        lax.fori_loop(0, CH, row_body, 0)
        pltpu.sync_copy(
            out_v, out_hbm.at[pl.ds(wid * ROWS_PER_W + c * CH, CH)])
        return carry

    lax.fori_loop(0, NCH, chunk_body, 0)


@functools.partial(
    pl.kernel,
    out_type=jax.ShapeDtypeStruct((B, K), jnp.float32),
    mesh=_mesh,
    compiler_params=pltpu.CompilerParams(use_tc_tiling_on_sc=False),
    scratch_types=_SC_SCRATCH,
)
def _combine_kernel(idx_hbm, sg_tab, sgb_hbm, ssum_hbm, out_hbm,
                    idx_v, rows_v, out_v, ss_v, bias_v, sem):
    wid = lax.axis_index("s") * NC + lax.axis_index("c")
    pltpu.sync_copy(sgb_hbm, bias_v)
    pltpu.sync_copy(
        idx_hbm.at[pl.ds(wid * (NCH * NDMA), NCH * NDMA)], idx_v)

    def chunk_body(c, carry):
        copies = []
        for j in range(NDMA):
            copies.append(pltpu.async_copy(
                sg_tab.at[idx_v.at[c * NDMA + j]],
                rows_v.at[pl.ds(j * 128, 128)], sem))
        pltpu.sync_copy(
            ssum_hbm.at[pl.ds(wid * ROWS_PER_W + c * CH, CH)], ss_v)
        for cp in copies:
            cp.wait()

        lanes = lax.broadcasted_iota(jnp.int32, (K,), 0)

        def lane_sum(x):
            for sh in (8, 4, 2, 1):
                x = x + x.at[lanes ^ sh].get(mode="promise_in_bounds")
            return x

        def row_body(r, rcarry):
            base = r * F
            g = bias_v[...]
            for f in range(F):
                g = g + rows_v[base + f, :]
            e = jnp.exp(ss_v[r, :])
            sig = 1.0 / (1.0 + jnp.exp(g))
            val = lane_sum(e * sig) / lane_sum(e)
            out_v[r, :] = val
            return rcarry

        lax.fori_loop(0, CH, row_body, 0)
        pltpu.sync_copy(
            out_v, out_hbm.at[pl.ds(wid * ROWS_PER_W + c * CH, CH)])
        return carry

    lax.fori_loop(0, NCH, chunk_body, 0)


def kernel(indices, softmax_W, sigmoid_W, softmax_bias, sigmoid_bias):
    col_off = (jnp.arange(F, dtype=jnp.int32) * ROW_PITCH)[None, :]
    perm = (indices % G) * 8 + indices // G
    flat_idx = (perm + col_off).reshape(B * F // 128, 128)
    sm_tab = _relayout(jnp.transpose(softmax_W, (0, 2, 1))).reshape(F * ROW_PITCH, K)
    s_sums = _sum_kernel(flat_idx, sm_tab, softmax_bias)
    sg_tab = _relayout(jnp.transpose(sigmoid_W, (0, 2, 1))).reshape(F * ROW_PITCH, K)
    logits = _combine_kernel(flat_idx, sg_tab, sigmoid_bias, s_sums)
    return logits[:, :1]
